# baseline (speedup 1.0000x reference)
import jax
import jax.numpy as jnp
from jax import lax
from jax.experimental import pallas as pl
from jax.experimental.pallas import tpu as pltpu
from jax.experimental.pallas import tpu_sc as plsc

VOCAB = 1000000
D = 64
DP = 128
BATCH = 4096
HIST = 200
B = BATCH * HIST

_info = plsc.get_sparse_core_info()
NW = _info.num_cores * _info.num_subcores
B_PER_W = B // NW
CHUNK = 160
STEPS = B_PER_W // CHUNK


def _gather_body(table_hbm, idx_hbm, out_hbm,
                 idx0, idx1, rows0, rows1, pk0, pk1,
                 gs0, gs1, ss0, ss1):
    wid = lax.axis_index("s") * _info.num_cores + lax.axis_index("c")
    base = wid * B_PER_W
    idx = (idx0, idx1)
    rows = (rows0, rows1)
    packed = (pk0, pk1)
    gsem = (gs0, gs1)
    ssem = (ss0, ss1)

    def g_start(i, b):
        pltpu.sync_copy(idx_hbm.at[pl.ds(base + i * CHUNK, CHUNK)], idx[b])
        pltpu.async_copy(table_hbm.at[idx[b]], rows[b], gsem[b])

    def g_wait(b):
        pltpu.make_async_copy(
            table_hbm.at[pl.ds(0, CHUNK)], rows[b], gsem[b]).wait()

    def compact(b):
        def row(r, _):
            for k in range(4):
                packed[b][r, pl.ds(k * 16, 16)] = rows[b][r, pl.ds(k * 16, 16)]
            return None

        lax.fori_loop(0, CHUNK, row, None)

    def s_start(i, b):
        pltpu.async_copy(
            packed[b], out_hbm.at[pl.ds(base + i * CHUNK, CHUNK)], ssem[b])

    def s_wait(b):
        pltpu.make_async_copy(
            packed[b], out_hbm.at[pl.ds(base, CHUNK)], ssem[b]).wait()

    g_start(0, 0)
    g_wait(0)
    compact(0)
    s_start(0, 0)
    g_start(1, 1)
    g_wait(1)
    compact(1)
    s_start(1, 1)
    g_start(2, 0)

    def pair(k, _):
        for off in (0, 1):
            i = 2 + 2 * k + off
            b = off
            g_wait(b)
            s_wait(b)
            compact(b)
            s_start(i, b)
            g_start(i + 1, 1 - b)
        return None

    lax.fori_loop(0, (STEPS - 4) // 2, pair, None)

    g_wait(0)
    s_wait(0)
    compact(0)
    s_start(STEPS - 2, 0)
    g_start(STEPS - 1, 1)
    g_wait(1)
    s_wait(1)
    compact(1)
    s_start(STEPS - 1, 1)
    s_wait(0)
    s_wait(1)


_gather_call = pl.kernel(
    _gather_body,
    mesh=plsc.VectorSubcoreMesh(core_axis_name="c", subcore_axis_name="s"),
    out_type=jax.ShapeDtypeStruct((B, D), jnp.float32),
    scratch_types=[
        pltpu.VMEM((CHUNK,), jnp.int32),
        pltpu.VMEM((CHUNK,), jnp.int32),
        pltpu.VMEM((CHUNK, DP), jnp.float32),
        pltpu.VMEM((CHUNK, DP), jnp.float32),
        pltpu.VMEM((CHUNK, D), jnp.float32),
        pltpu.VMEM((CHUNK, D), jnp.float32),
        pltpu.SemaphoreType.DMA,
        pltpu.SemaphoreType.DMA,
        pltpu.SemaphoreType.DMA,
        pltpu.SemaphoreType.DMA,
    ],
    compiler_params=pltpu.CompilerParams(use_tc_tiling_on_sc=True),
)


def kernel(x, embeddings):
    idx = x.reshape(B).astype(jnp.int32)
    t128 = jnp.pad(embeddings, ((0, 0), (0, DP - D)))
    out = _gather_call(t128, idx)
    return out.reshape(BATCH, HIST, D)
